# Initial kernel scaffold; baseline (speedup 1.0000x reference)
import jax
import jax.numpy as jnp
from jax.experimental import pallas as pl


def kernel(x, edge_index, W, gamma, beta):
    raise NotImplementedError("write your pallas kernel here")


# speedup vs baseline: 3.1363x; 3.1363x over previous
import functools

import jax
import jax.numpy as jnp
from jax import lax
from jax.experimental import pallas as pl
from jax.experimental.pallas import tpu as pltpu
from jax.experimental.pallas import tpu_sc as plsc

C = 128
OUT = 128
N = 10000
K = 16
N_PAD = 10240
NW = 32
CPT = OUT // NW
CHUNK = 2048
N_CHUNKS = N_PAD // CHUNK
COLS = 1024
NKF = float(N * K)


def _prep_body(w_ref, x_ref, a_ref, b_ref):
    w1 = w_ref[:, :C]
    w2 = w_ref[:, C:]
    xb = x_ref[...]
    a_ref[...] = jnp.dot(w1 + w2, xb, preferred_element_type=jnp.float32)
    b_ref[...] = jnp.dot(w2, xb, preferred_element_type=jnp.float32)


def _prep(w, xp):
    grid = N_PAD // COLS
    return pl.pallas_call(
        _prep_body,
        grid=(grid,),
        in_specs=[
            pl.BlockSpec((OUT, 2 * C), lambda i: (0, 0)),
            pl.BlockSpec((C, COLS), lambda i: (0, i)),
        ],
        out_specs=[
            pl.BlockSpec((OUT, COLS), lambda i: (0, i)),
            pl.BlockSpec((OUT, COLS), lambda i: (0, i)),
        ],
        out_shape=[
            jax.ShapeDtypeStruct((OUT, N_PAD), jnp.float32),
            jax.ShapeDtypeStruct((OUT, N_PAD), jnp.float32),
        ],
    )(w, xp)


def _sc_body(bm_hbm, idxt_hbm, minb_hbm, s_hbm, q_hbm, t0, t1, t2, t3, idxbuf, mnb, smb, qqb):
    wid = lax.axis_index("s") * 2 + lax.axis_index("c")
    cb = wid * CPT
    tbls = [t0, t1, t2, t3]
    for c in range(CPT):
        pltpu.sync_copy(bm_hbm.at[cb + c, :], tbls[c])

    lanes = lax.iota(jnp.int32, 16)

    for ch in range(N_CHUNKS):
        base = ch * CHUNK
        pltpu.sync_copy(idxt_hbm.at[:, pl.ds(base, CHUNK)], idxbuf)

        def group(g, _, base=base):
            off = g * 16
            idxvs = [idxbuf[k, pl.ds(off, 16)] for k in range(K)]
            mask = (lanes + (base + off)) < N
            for c in range(CPT):
                v0 = plsc.load_gather(tbls[c], [idxvs[0]])
                mn = v0
                sm = v0
                qq = v0 * v0
                for k in range(1, K):
                    v = plsc.load_gather(tbls[c], [idxvs[k]])
                    mn = jnp.minimum(mn, v)
                    sm = sm + v
                    qq = qq + v * v
                mnb[c, pl.ds(off, 16)] = mn
                smb[c, pl.ds(off, 16)] = jnp.where(mask, sm, 0.0)
                qqb[c, pl.ds(off, 16)] = jnp.where(mask, qq, 0.0)
            return 0

        lax.fori_loop(0, CHUNK // 16, group, 0)

        pltpu.sync_copy(mnb, minb_hbm.at[pl.ds(cb, CPT), pl.ds(base, CHUNK)])
        pltpu.sync_copy(smb, s_hbm.at[pl.ds(cb, CPT), pl.ds(base, CHUNK)])
        pltpu.sync_copy(qqb, q_hbm.at[pl.ds(cb, CPT), pl.ds(base, CHUNK)])


def _sc_gather(bm, idxt):
    mesh = plsc.VectorSubcoreMesh(core_axis_name="c", subcore_axis_name="s")
    f = pl.kernel(
        _sc_body,
        out_type=[
            jax.ShapeDtypeStruct((OUT, N_PAD), jnp.float32),
            jax.ShapeDtypeStruct((OUT, N_PAD), jnp.float32),
            jax.ShapeDtypeStruct((OUT, N_PAD), jnp.float32),
        ],
        mesh=mesh,
        compiler_params=pltpu.CompilerParams(needs_layout_passes=False),
        scratch_types=[
            pltpu.VMEM((N_PAD,), jnp.float32),
            pltpu.VMEM((N_PAD,), jnp.float32),
            pltpu.VMEM((N_PAD,), jnp.float32),
            pltpu.VMEM((N_PAD,), jnp.float32),
            pltpu.VMEM((K, CHUNK), jnp.int32),
            pltpu.VMEM((CPT, CHUNK), jnp.float32),
            pltpu.VMEM((CPT, CHUNK), jnp.float32),
            pltpu.VMEM((CPT, CHUNK), jnp.float32),
        ],
    )
    return f(bm, idxt)


def _stats_body(a_ref, s_ref, q_ref, s1_ref, s2_ref):
    i = pl.program_id(0)
    a = a_ref[...]
    s = s_ref[...]
    q = q_ref[...]
    p1 = K * jnp.sum(a, axis=1, keepdims=True) - jnp.sum(s, axis=1, keepdims=True)
    p2 = (
        K * jnp.sum(a * a, axis=1, keepdims=True)
        - 2.0 * jnp.sum(a * s, axis=1, keepdims=True)
        + jnp.sum(q, axis=1, keepdims=True)
    )

    @pl.when(i == 0)
    def _():
        s1_ref[...] = p1
        s2_ref[...] = p2

    @pl.when(i != 0)
    def _():
        s1_ref[...] += p1
        s2_ref[...] += p2


def _stats(a, s, q):
    grid = N_PAD // COLS
    return pl.pallas_call(
        _stats_body,
        grid=(grid,),
        in_specs=[
            pl.BlockSpec((OUT, COLS), lambda i: (0, i)),
            pl.BlockSpec((OUT, COLS), lambda i: (0, i)),
            pl.BlockSpec((OUT, COLS), lambda i: (0, i)),
        ],
        out_specs=[
            pl.BlockSpec((OUT, 1), lambda i: (0, 0)),
            pl.BlockSpec((OUT, 1), lambda i: (0, 0)),
        ],
        out_shape=[
            jax.ShapeDtypeStruct((OUT, 1), jnp.float32),
            jax.ShapeDtypeStruct((OUT, 1), jnp.float32),
        ],
    )(a, s, q)


def _final_body(a_ref, mb_ref, s1_ref, s2_ref, g_ref, b_ref, o_ref):
    mean = s1_ref[...] * (1.0 / NKF)
    e2 = s2_ref[...] * (1.0 / NKF)
    var = e2 - mean * mean
    rstd = lax.rsqrt(var + 1e-5)
    scale = g_ref[...] * rstd
    shift = b_ref[...] - mean * scale
    h = (a_ref[...] - mb_ref[...]) * scale + shift
    o_ref[...] = jnp.where(h >= 0.0, h, 0.2 * h)


def _final(a, minb, s1, s2, gamma, beta):
    grid = N_PAD // COLS
    return pl.pallas_call(
        _final_body,
        grid=(grid,),
        in_specs=[
            pl.BlockSpec((OUT, COLS), lambda i: (0, i)),
            pl.BlockSpec((OUT, COLS), lambda i: (0, i)),
            pl.BlockSpec((OUT, 1), lambda i: (0, 0)),
            pl.BlockSpec((OUT, 1), lambda i: (0, 0)),
            pl.BlockSpec((OUT, 1), lambda i: (0, 0)),
            pl.BlockSpec((OUT, 1), lambda i: (0, 0)),
        ],
        out_specs=pl.BlockSpec((OUT, COLS), lambda i: (0, i)),
        out_shape=jax.ShapeDtypeStruct((OUT, N), jnp.float32),
    )(a, minb, s1, s2, gamma, beta)


@jax.jit
def kernel(x, edge_index, W, gamma, beta):
    xp = jnp.pad(x.reshape(C, N), ((0, 0), (0, N_PAD - N)))
    idxt = jnp.pad(edge_index.reshape(N, K).T, ((0, 0), (0, N_PAD - N)))
    a, bm = _prep(W, xp)
    minb, s, q = _sc_gather(bm, idxt)
    s1, s2 = _stats(a, s, q)
    out = _final(a, minb, s1, s2, gamma.reshape(OUT, 1), beta.reshape(OUT, 1))
    return out.reshape(1, OUT, N, 1)


# speedup vs baseline: 4.2813x; 1.3650x over previous
import functools

import jax
import jax.numpy as jnp
from jax import lax
from jax.experimental import pallas as pl
from jax.experimental.pallas import tpu as pltpu
from jax.experimental.pallas import tpu_sc as plsc

C = 128
OUT = 128
N = 10000
K = 16
N_PAD = 10240
NW = 32
CPT = OUT // NW
CHUNK = 1024
N_CHUNKS = N_PAD // CHUNK
GROUPS = CHUNK // 16
COLS = 1024
NKF = float(N * K)


def _prep_body(w_ref, x_ref, a_ref, b_ref, sa_ref, sa2_ref):
    i = pl.program_id(0)
    w1 = w_ref[:, :C]
    w2 = w_ref[:, C:]
    xb = x_ref[...]
    a = jnp.dot(w1 + w2, xb, preferred_element_type=jnp.float32)
    a_ref[...] = a
    b_ref[...] = jnp.dot(w2, xb, preferred_element_type=jnp.float32)
    p1 = jnp.sum(a, axis=1, keepdims=True)
    p2 = jnp.sum(a * a, axis=1, keepdims=True)

    @pl.when(i == 0)
    def _():
        sa_ref[...] = p1
        sa2_ref[...] = p2

    @pl.when(i != 0)
    def _():
        sa_ref[...] += p1
        sa2_ref[...] += p2


def _prep(w, xp):
    grid = N_PAD // COLS
    return pl.pallas_call(
        _prep_body,
        grid=(grid,),
        in_specs=[
            pl.BlockSpec((OUT, 2 * C), lambda i: (0, 0)),
            pl.BlockSpec((C, COLS), lambda i: (0, i)),
        ],
        out_specs=[
            pl.BlockSpec((OUT, COLS), lambda i: (0, i)),
            pl.BlockSpec((OUT, COLS), lambda i: (0, i)),
            pl.BlockSpec((OUT, 1), lambda i: (0, 0)),
            pl.BlockSpec((OUT, 1), lambda i: (0, 0)),
        ],
        out_shape=[
            jax.ShapeDtypeStruct((OUT, N_PAD), jnp.float32),
            jax.ShapeDtypeStruct((OUT, N_PAD), jnp.float32),
            jax.ShapeDtypeStruct((OUT, 1), jnp.float32),
            jax.ShapeDtypeStruct((OUT, 1), jnp.float32),
        ],
    )(w, xp)


def _sc_body(
    bm_hbm, a_hbm, idxt_hbm,
    minb_hbm, ps_hbm, pas_hbm, pq_hbm,
    t0, t1, t2, t3,
    idx0, idx1, a0, a1, mnb0, mnb1, psb,
    sem_i0, sem_i1, sem_a0, sem_a1, sem_o0, sem_o1,
):
    wid = lax.axis_index("s") * 2 + lax.axis_index("c")
    cb = wid * CPT
    tbls = [t0, t1, t2, t3]
    idxb = [idx0, idx1]
    ab = [a0, a1]
    mnbb = [mnb0, mnb1]
    sem_i = [sem_i0, sem_i1]
    sem_a = [sem_a0, sem_a1]
    sem_o = [sem_o0, sem_o1]

    for c in range(CPT):
        pltpu.sync_copy(bm_hbm.at[cb + c, :], tbls[c])

    def in_copies(ch):
        p = ch % 2
        base = ch * CHUNK
        di = pltpu.make_async_copy(idxt_hbm.at[:, pl.ds(base, CHUNK)], idxb[p], sem_i[p])
        da = pltpu.make_async_copy(a_hbm.at[pl.ds(cb, CPT), pl.ds(base, CHUNK)], ab[p], sem_a[p])
        return di, da

    def out_copy(ch):
        p = ch % 2
        base = ch * CHUNK
        return pltpu.make_async_copy(
            mnbb[p], minb_hbm.at[pl.ds(cb, CPT), pl.ds(base, CHUNK)], sem_o[p]
        )

    lanes = lax.iota(jnp.int32, 16)
    zero = jnp.zeros((16,), jnp.float32)
    accs = (zero,) * (3 * CPT)

    d0 = in_copies(0)
    d0[0].start()
    d0[1].start()

    for ch in range(N_CHUNKS):
        p = ch % 2
        base = ch * CHUNK
        if ch + 1 < N_CHUNKS:
            dn = in_copies(ch + 1)
            dn[0].start()
            dn[1].start()
        di, da = in_copies(ch)
        di.wait()
        da.wait()
        if ch >= 2:
            out_copy(ch - 2).wait()

        idxr = idxb[p]
        ar = ab[p]
        mr = mnbb[p]

        def group(g, accs, base=base, idxr=idxr, ar=ar, mr=mr):
            off = g * 16
            accs = list(accs)
            idxvs = [idxr[k, pl.ds(off, 16)] for k in range(K)]
            mask = (lanes + (base + off)) < N
            for c in range(CPT):
                v0 = plsc.load_gather(tbls[c], [idxvs[0]])
                mn = v0
                sm = v0
                qq = v0 * v0
                for k in range(1, K):
                    v = plsc.load_gather(tbls[c], [idxvs[k]])
                    mn = jnp.minimum(mn, v)
                    sm = sm + v
                    qq = qq + v * v
                mr[c, pl.ds(off, 16)] = mn
                av = ar[c, pl.ds(off, 16)]
                smm = jnp.where(mask, sm, 0.0)
                qqm = jnp.where(mask, qq, 0.0)
                accs[3 * c] = accs[3 * c] + smm
                accs[3 * c + 1] = accs[3 * c + 1] + av * smm
                accs[3 * c + 2] = accs[3 * c + 2] + qqm
            return tuple(accs)

        accs = lax.fori_loop(0, GROUPS, group, accs)
        out_copy(ch).start()

    out_copy(N_CHUNKS - 2).wait()
    out_copy(N_CHUNKS - 1).wait()

    for c in range(CPT):
        psb[c, pl.ds(0, 16)] = accs[3 * c]
    pltpu.sync_copy(psb, ps_hbm.at[pl.ds(cb, CPT), :])
    for c in range(CPT):
        psb[c, pl.ds(0, 16)] = accs[3 * c + 1]
    pltpu.sync_copy(psb, pas_hbm.at[pl.ds(cb, CPT), :])
    for c in range(CPT):
        psb[c, pl.ds(0, 16)] = accs[3 * c + 2]
    pltpu.sync_copy(psb, pq_hbm.at[pl.ds(cb, CPT), :])


def _sc_gather(bm, a, idxt):
    mesh = plsc.VectorSubcoreMesh(core_axis_name="c", subcore_axis_name="s")
    f = pl.kernel(
        _sc_body,
        out_type=[
            jax.ShapeDtypeStruct((OUT, N_PAD), jnp.float32),
            jax.ShapeDtypeStruct((OUT, 16), jnp.float32),
            jax.ShapeDtypeStruct((OUT, 16), jnp.float32),
            jax.ShapeDtypeStruct((OUT, 16), jnp.float32),
        ],
        mesh=mesh,
        compiler_params=pltpu.CompilerParams(needs_layout_passes=False),
        scratch_types=[
            pltpu.VMEM((N_PAD,), jnp.float32),
            pltpu.VMEM((N_PAD,), jnp.float32),
            pltpu.VMEM((N_PAD,), jnp.float32),
            pltpu.VMEM((N_PAD,), jnp.float32),
            pltpu.VMEM((K, CHUNK), jnp.int32),
            pltpu.VMEM((K, CHUNK), jnp.int32),
            pltpu.VMEM((CPT, CHUNK), jnp.float32),
            pltpu.VMEM((CPT, CHUNK), jnp.float32),
            pltpu.VMEM((CPT, CHUNK), jnp.float32),
            pltpu.VMEM((CPT, CHUNK), jnp.float32),
            pltpu.VMEM((CPT, 16), jnp.float32),
            pltpu.SemaphoreType.DMA,
            pltpu.SemaphoreType.DMA,
            pltpu.SemaphoreType.DMA,
            pltpu.SemaphoreType.DMA,
            pltpu.SemaphoreType.DMA,
            pltpu.SemaphoreType.DMA,
        ],
    )
    return f(bm, a, idxt)


def _final_body(a_ref, mb_ref, sa_ref, sa2_ref, ps_ref, pas_ref, pq_ref, g_ref, b_ref, o_ref):
    s1 = K * sa_ref[...] - jnp.sum(ps_ref[...], axis=1, keepdims=True)
    s2 = (
        K * sa2_ref[...]
        - 2.0 * jnp.sum(pas_ref[...], axis=1, keepdims=True)
        + jnp.sum(pq_ref[...], axis=1, keepdims=True)
    )
    mean = s1 * (1.0 / NKF)
    e2 = s2 * (1.0 / NKF)
    var = e2 - mean * mean
    rstd = lax.rsqrt(var + 1e-5)
    scale = g_ref[...] * rstd
    shift = b_ref[...] - mean * scale
    h = (a_ref[...] - mb_ref[...]) * scale + shift
    o_ref[...] = jnp.where(h >= 0.0, h, 0.2 * h)


def _final(a, minb, sa, sa2, ps, pas, pq, gamma, beta):
    grid = N_PAD // COLS
    vec = pl.BlockSpec((OUT, 1), lambda i: (0, 0))
    part = pl.BlockSpec((OUT, 16), lambda i: (0, 0))
    return pl.pallas_call(
        _final_body,
        grid=(grid,),
        in_specs=[
            pl.BlockSpec((OUT, COLS), lambda i: (0, i)),
            pl.BlockSpec((OUT, COLS), lambda i: (0, i)),
            vec, vec, part, part, part, vec, vec,
        ],
        out_specs=pl.BlockSpec((OUT, COLS), lambda i: (0, i)),
        out_shape=jax.ShapeDtypeStruct((OUT, N), jnp.float32),
    )(a, minb, sa, sa2, ps, pas, pq, gamma, beta)


@jax.jit
def kernel(x, edge_index, W, gamma, beta):
    xp = jnp.pad(x.reshape(C, N), ((0, 0), (0, N_PAD - N)))
    idxt = jnp.pad(edge_index.reshape(N, K).T, ((0, 0), (0, N_PAD - N)))
    a, bm, sa, sa2 = _prep(W, xp)
    minb, ps, pas, pq = _sc_gather(bm, a, idxt)
    out = _final(a, minb, sa, sa2, ps, pas, pq, gamma.reshape(OUT, 1), beta.reshape(OUT, 1))
    return out.reshape(1, OUT, N, 1)
